# baseline (speedup 1.0000x reference)
import functools

import jax
import jax.numpy as jnp
import numpy as np
from jax import lax
from jax.experimental import pallas as pl
from jax.experimental.pallas import tpu as pltpu
from jax.experimental.pallas import tpu_sc as plsc

_NC, _NS = 2, 16
_NW = _NC * _NS
_NB = 4
_C = 32
_MCHUNK = 32
_PW = _C * _MCHUNK
_HALF = _NW * _PW
_ROWS = 2 * _HALF
_D = 768


def _morton_tables() -> tuple[np.ndarray, np.ndarray]:
    n = 1024
    k = np.arange(n, dtype=np.int64)
    row = np.zeros(n, np.int64)
    col = np.zeros(n, np.int64)
    for b in range(5):
        col |= ((k >> (2 * b)) & 1) << b
        row |= ((k >> (2 * b + 1)) & 1) << b
    mask = row * 32 + col
    inv = np.argsort(mask)
    return mask, inv


def _index_tables() -> tuple[np.ndarray, np.ndarray]:
    mask, inv = _morton_tables()
    r = np.arange(_HALF, dtype=np.int64)
    g = (r >> 10) * 1024 + mask[r & 1023]
    rs = _HALF + r
    s = (rs >> 10) * 1024 + inv[rs & 1023]
    shape = (_NW, _MCHUNK, _C)
    return g.astype(np.int32).reshape(shape), s.astype(np.int32).reshape(shape)


_IDX_G_NP, _IDX_S_NP = _index_tables()


@functools.cache
def _build_zorder_sc():
    mesh = plsc.VectorSubcoreMesh(core_axis_name="c", subcore_axis_name="s")

    @functools.partial(
        pl.kernel,
        mesh=mesh,
        out_type=jax.ShapeDtypeStruct((_ROWS, _D), jnp.float32),
        scratch_types=(
            [pltpu.VMEM((_MCHUNK, _C), jnp.int32)] * 2
            + [pltpu.VMEM((_C, _D), jnp.float32)] * _NB
            + [pltpu.SemaphoreType.DMA] * (2 * _NB)
        ),
    )
    def _zorder_sc(table, idx_g, idx_s, out, idxg_v, idxs_v, *rest):
        bufs, sgs, sws = rest[:_NB], rest[_NB:2 * _NB], rest[2 * _NB:]
        wid = lax.axis_index("s") * _NC + lax.axis_index("c")
        gbase = wid * _PW
        sbase = _HALF + wid * _PW
        pltpu.sync_copy(idx_g.at[wid], idxg_v)
        pltpu.sync_copy(idx_s.at[wid], idxs_v)

        def read_src(b, mch):
            if b % 2 == 0:
                return table.at[idxg_v.at[mch]]
            return table.at[pl.ds(sbase + mch * _C, _C)]

        def read_start(b, mch):
            pltpu.async_copy(read_src(b, mch), bufs[b], sgs[b])

        def drain_and_write(b, mch):
            pltpu.make_async_copy(read_src(b, mch), bufs[b], sgs[b]).wait()
            if b % 2 == 0:
                pltpu.async_copy(bufs[b], out.at[pl.ds(gbase + mch * _C, _C)], sws[b]).wait()
            else:
                pltpu.async_copy(bufs[b], out.at[idxs_v.at[mch]], sws[b]).wait()

        for b in range(_NB):
            read_start(b, b // 2)

        def body(i, carry):
            for b in range(_NB):
                mch = i * 2 + b // 2
                drain_and_write(b, mch)
                read_start(b, mch + 2)
            return carry

        lax.fori_loop(0, _MCHUNK // 2 - 1, body, 0)

        for b in range(_NB):
            drain_and_write(b, _MCHUNK - 2 + b // 2)

    return _zorder_sc


def kernel(inputs):
    b, w, h, c = inputs.shape
    flat = inputs.reshape(_ROWS, _D)
    out = _build_zorder_sc()(flat, jnp.asarray(_IDX_G_NP), jnp.asarray(_IDX_S_NP))
    return out.reshape(b, w * h, c)


# speedup vs baseline: 1.0065x; 1.0065x over previous
import functools

import jax
import jax.numpy as jnp
import numpy as np
from jax import lax
from jax.experimental import pallas as pl
from jax.experimental.pallas import tpu as pltpu
from jax.experimental.pallas import tpu_sc as plsc

_NC, _NS = 2, 16
_NW = _NC * _NS
_NB = 2
_C = 64
_NCHUNK = 32
_PW = _C * _NCHUNK
_ROWS = _NW * _PW
_D = 768


def _gather_index_table() -> np.ndarray:
    n = 1024
    k = np.arange(n, dtype=np.int64)
    row = np.zeros(n, np.int64)
    col = np.zeros(n, np.int64)
    for b in range(5):
        col |= ((k >> (2 * b)) & 1) << b
        row |= ((k >> (2 * b + 1)) & 1) << b
    mask = row * 32 + col
    r = np.arange(_ROWS, dtype=np.int64)
    g = (r >> 10) * 1024 + mask[r & 1023]
    return g.astype(np.int32).reshape(_NW, _NCHUNK, _C)


_IDX_NP = _gather_index_table()


@functools.cache
def _build_zorder_sc():
    mesh = plsc.VectorSubcoreMesh(core_axis_name="c", subcore_axis_name="s")

    @functools.partial(
        pl.kernel,
        mesh=mesh,
        out_type=jax.ShapeDtypeStruct((_ROWS, _D), jnp.float32),
        scratch_types=(
            [pltpu.VMEM((_NCHUNK, _C), jnp.int32)]
            + [pltpu.VMEM((_C, _D), jnp.float32)] * _NB
            + [pltpu.SemaphoreType.DMA] * (2 * _NB)
        ),
    )
    def _zorder_sc(table, idxs, out, idx_v, *rest):
        bufs, sgs, sws = rest[:_NB], rest[_NB:2 * _NB], rest[2 * _NB:]
        wid = lax.axis_index("s") * _NC + lax.axis_index("c")
        base = wid * _PW
        pltpu.sync_copy(idxs.at[wid], idx_v)

        for b in range(_NB):
            pltpu.async_copy(table.at[idx_v.at[b]], bufs[b], sgs[b])

        def body(i, carry):
            for b in range(_NB):
                ch = i * _NB + b
                pltpu.make_async_copy(table.at[idx_v.at[ch]], bufs[b], sgs[b]).wait()
                pltpu.async_copy(bufs[b], out.at[pl.ds(base + ch * _C, _C)], sws[b]).wait()
                pltpu.async_copy(table.at[idx_v.at[ch + _NB]], bufs[b], sgs[b])
            return carry

        lax.fori_loop(0, _NCHUNK // _NB - 1, body, 0)

        for b in range(_NB):
            ch = _NCHUNK - _NB + b
            pltpu.make_async_copy(table.at[idx_v.at[ch]], bufs[b], sgs[b]).wait()
            pltpu.async_copy(bufs[b], out.at[pl.ds(base + ch * _C, _C)], sws[b]).wait()

    return _zorder_sc


def kernel(inputs):
    b, w, h, c = inputs.shape
    flat = inputs.reshape(_ROWS, _D)
    out = _build_zorder_sc()(flat, jnp.asarray(_IDX_NP))
    return out.reshape(b, w * h, c)
